# baseline (speedup 1.0000x reference)
import functools

import numpy as np
import jax
import jax.numpy as jnp
from jax import lax
from jax.experimental import pallas as pl
from jax.experimental.pallas import tpu as pltpu
from jax.experimental.pallas import tpu_sc as plsc

_B = 1024
_LANES = 128
_NT = 32
_SB = 16
_NB = _B // _SB
_PK = 200
_Q2B = 100
_NV = 208


def _conv_mat_indices(cout, cin, hin, win, k):
    hout, wout = hin - k + 1, win - k + 1
    o, i, j, c, di, dj = np.meshgrid(
        np.arange(cout), np.arange(hout), np.arange(wout),
        np.arange(cin), np.arange(k), np.arange(k), indexing="ij")
    rows = (o * hout + i) * wout + j
    cols = (c * hin + (i + di)) * win + (j + dj)
    widx = ((o * cin + c) * k + di) * k + dj
    return rows.ravel(), cols.ravel(), widx.ravel()

def _conv_gather_map(cout, cin, hin, win, k):
    hout, wout = hin - k + 1, win - k + 1
    rows, cols, widx = _conv_mat_indices(cout, cin, hin, win, k)
    m = np.full((cout * hout * wout, cin * hin * win),
                cout * cin * k * k, dtype=np.int32)
    m[rows, cols] = widx
    return m

_G1 = _conv_gather_map(6, 1, 28, 28, 5)
_G2 = _conv_gather_map(16, 6, 12, 12, 5)


def _aug(mat, bias, pad_to):
    n, k = mat.shape
    return jnp.concatenate(
        [mat, bias.reshape(n, 1), jnp.zeros((n, pad_to - k - 1), mat.dtype)],
        axis=1)


def _ones_pad(v, pad_to):
    k, L = v.shape
    return jnp.concatenate(
        [v, jnp.ones((1, L), v.dtype), jnp.zeros((pad_to - k - 1, L), v.dtype)],
        axis=0)


def _pool_relu(h, c, s):
    t = h.reshape(c, 2 * s, s, 2, _LANES)
    t = jnp.max(t, axis=3)
    t = t.reshape(c, s, 2, s, _LANES)
    t = jnp.max(t, axis=2)
    return jnp.maximum(t, 0.0).reshape(c * s * s, _LANES)


def _net_body(xr, m1r, m2r, f1r, f2r, f3r, outr):
    probs = []
    for d in range(4):
        Xa = _ones_pad(xr[784 * d:784 * (d + 1), :], 792)
        h1 = jnp.dot(m1r[...], Xa, preferred_element_type=jnp.float32)
        p1 = _pool_relu(h1, 6, 12)
        h2 = jnp.dot(m2r[...], _ones_pad(p1, 872),
                     preferred_element_type=jnp.float32)
        p2 = _pool_relu(h2, 16, 4)
        h = jnp.maximum(jnp.dot(f1r[...], _ones_pad(p2, 264),
                                preferred_element_type=jnp.float32), 0.0)
        h = jnp.maximum(jnp.dot(f2r[...], _ones_pad(h, 128),
                                preferred_element_type=jnp.float32), 0.0)
        lg = jnp.dot(f3r[...], _ones_pad(h, 88),
                     preferred_element_type=jnp.float32)
        m = jnp.max(lg, axis=0, keepdims=True)
        e = jnp.exp(lg - m)
        probs.append(e / jnp.sum(e, axis=0, keepdims=True))
    q1 = (probs[0][:, None, :] * probs[1][None, :, :]).reshape(100, _LANES)
    q2 = (probs[2][:, None, :] * probs[3][None, :, :]).reshape(100, _LANES)
    outr[...] = jnp.concatenate([q1, q2], axis=0)


def _run_net(X2, W1, b1, W2, b2, Wf1, bf1, Wf2, bf2, Wf3, bf3):
    w1e = jnp.concatenate([W1.ravel(), jnp.zeros((1,), jnp.float32)])
    m1 = _aug(w1e[_G1], jnp.repeat(b1, 576), 792)
    w2e = jnp.concatenate([W2.ravel(), jnp.zeros((1,), jnp.float32)])
    m2 = _aug(w2e[_G2], jnp.repeat(b2, 64), 872)
    f1 = _aug(Wf1, bf1, 264)
    f2 = _aug(Wf2, bf2, 128)
    f3 = _aug(Wf3, bf3, 88)
    full = lambda shape: pl.BlockSpec(shape, lambda i: (0,) * len(shape))
    return pl.pallas_call(
        _net_body,
        grid=(_B // _LANES,),
        in_specs=[
            pl.BlockSpec((3136, _LANES), lambda i: (0, i)),
            full((3456, 792)), full((1024, 872)),
            full((120, 264)), full((84, 128)), full((10, 88)),
        ],
        out_specs=pl.BlockSpec((_PK, _LANES), lambda i: (0, i)),
        out_shape=jax.ShapeDtypeStruct((_PK, _B), jnp.float32),
    )(X2, m1, m2, f1, f2, f3)


def _sc_body(d_hbm, out_hbm, dv, ov):
    wid = lax.axis_index("s") * 2 + lax.axis_index("c")
    for p in range(_NB // _NT):
        blk = wid * (_NB // _NT) + p
        pltpu.sync_copy(d_hbm.at[blk], dv)

        def vbody(v, _):
            jlo = jnp.maximum(0, v - 99)
            jhi1 = jnp.minimum(99, v) + 1

            def insert(j, M):
                x = dv[j] * dv[_Q2B + v - j]
                out = []
                for r in range(8):
                    mr = M[r]
                    out.append(jnp.maximum(mr, x))
                    x = jnp.minimum(mr, x)
                return tuple(out)

            M = lax.fori_loop(jlo, jhi1, insert,
                              (jnp.zeros((_SB,), jnp.float32),) * 8)
            s = M[0]
            for r in range(1, 8):
                s = s + M[r]
            ov[v] = s
            return 0

        lax.fori_loop(0, _NV, vbody, 0)
        pltpu.sync_copy(ov, out_hbm.at[blk])


def _run_sc(packed3):
    mesh = plsc.VectorSubcoreMesh(core_axis_name="c", subcore_axis_name="s")
    return pl.kernel(
        _sc_body,
        out_type=jax.ShapeDtypeStruct((_NB, _NV, _SB), jnp.float32),
        mesh=mesh,
        scratch_types=[
            pltpu.VMEM((_PK, _SB), jnp.float32),
            pltpu.VMEM((_NV, _SB), jnp.float32),
        ],
    )(packed3)


def _bucket_body(inr, outr):
    vv = lax.broadcasted_iota(jnp.int32, (_NV, 24), 0).astype(jnp.float32)
    cc = lax.broadcasted_iota(jnp.int32, (_NV, 24), 1).astype(jnp.float32)
    eqf = lambda a, b: jnp.maximum(0.0, 1.0 - jnp.abs(a - b))
    v10 = jnp.floor(vv / 10.0)
    d0 = vv - 10.0 * v10
    d2 = jnp.floor(vv / 100.0)
    d1 = v10 - 10.0 * d2
    ic = jnp.floor(cc / 10.0)
    digit = d0 * eqf(ic, 0.0) + d1 * eqf(ic, 1.0) + d2 * eqf(ic, 2.0)
    valid = jnp.clip(199.0 - vv, 0.0, 1.0)
    mask = eqf(digit, cc - 10.0 * ic) * valid
    outr[...] = jnp.dot(inr[...], mask, preferred_element_type=jnp.float32)


def _run_buckets(rows):
    return pl.pallas_call(
        _bucket_body,
        grid=(_B // _LANES,),
        in_specs=[pl.BlockSpec((_LANES, _NV), lambda i: (i, 0))],
        out_specs=pl.BlockSpec((_LANES, 24), lambda i: (i, 0)),
        out_shape=jax.ShapeDtypeStruct((_B, 24), jnp.float32),
    )(rows)


def kernel(x, W1, b1, W2, b2, Wf1, bf1, Wf2, bf2, Wf3, bf3):
    B = x.shape[0]
    X2 = x.reshape(B, 4 * 784).T
    packed = _run_net(X2, W1, b1, W2, b2, Wf1, bf1, Wf2, bf2, Wf3, bf3)
    packed3 = packed.reshape(_PK, _NB, _SB).transpose(1, 0, 2)
    rows = _run_sc(packed3).transpose(0, 2, 1).reshape(B, _NV)
    res = _run_buckets(rows)
    return (res[:, 0:10], res[:, 10:20], res[:, 20:22])


# speedup vs baseline: 84.7300x; 84.7300x over previous
import functools

import numpy as np
import jax
import jax.numpy as jnp
from jax import lax
from jax.experimental import pallas as pl
from jax.experimental.pallas import tpu as pltpu
from jax.experimental.pallas import tpu_sc as plsc

_B = 1024
_LANES = 128
_NT = 32
_SB = 16
_NB = _B // _SB
_PK = 200
_Q2B = 100
_NV = 208


def _conv_mat_indices(cout, cin, hin, win, k):
    hout, wout = hin - k + 1, win - k + 1
    o, i, j, c, di, dj = np.meshgrid(
        np.arange(cout), np.arange(hout), np.arange(wout),
        np.arange(cin), np.arange(k), np.arange(k), indexing="ij")
    rows = (o * hout + i) * wout + j
    cols = (c * hin + (i + di)) * win + (j + dj)
    widx = ((o * cin + c) * k + di) * k + dj
    return rows.ravel(), cols.ravel(), widx.ravel()

def _conv_sel(cin, hin, win, k):
    hout, wout = hin - k + 1, win - k + 1
    rows, cols, widx = _conv_mat_indices(1, cin, hin, win, k)
    s = np.zeros((cin * k * k, hout * wout * cin * hin * win), np.float32)
    s[widx, rows * (cin * hin * win) + cols] = 1.0
    return s

_S1 = _conv_sel(1, 28, 28, 5)
_S2 = _conv_sel(6, 12, 12, 5)


def _aug(mat, bias, pad_to):
    n, k = mat.shape
    return jnp.concatenate(
        [mat, bias.reshape(n, 1), jnp.zeros((n, pad_to - k - 1), mat.dtype)],
        axis=1)


def _ones_pad(v, pad_to):
    k, L = v.shape
    return jnp.concatenate(
        [v, jnp.ones((1, L), v.dtype), jnp.zeros((pad_to - k - 1, L), v.dtype)],
        axis=0)


def _pool_relu(h, c, s):
    t = h.reshape(c, 2 * s, s, 2, _LANES)
    t = jnp.max(t, axis=3)
    t = t.reshape(c, s, 2, s, _LANES)
    t = jnp.max(t, axis=2)
    return jnp.maximum(t, 0.0).reshape(c * s * s, _LANES)


def _net_body(xr, m1r, m2r, f1r, f2r, f3r, outr):
    probs = []
    for d in range(4):
        Xa = _ones_pad(xr[784 * d:784 * (d + 1), :], 792)
        h1 = jnp.dot(m1r[...], Xa, preferred_element_type=jnp.float32)
        p1 = _pool_relu(h1, 6, 12)
        h2 = jnp.dot(m2r[...], _ones_pad(p1, 872),
                     preferred_element_type=jnp.float32)
        p2 = _pool_relu(h2, 16, 4)
        h = jnp.maximum(jnp.dot(f1r[...], _ones_pad(p2, 264),
                                preferred_element_type=jnp.float32), 0.0)
        h = jnp.maximum(jnp.dot(f2r[...], _ones_pad(h, 128),
                                preferred_element_type=jnp.float32), 0.0)
        lg = jnp.dot(f3r[...], _ones_pad(h, 88),
                     preferred_element_type=jnp.float32)
        m = jnp.max(lg, axis=0, keepdims=True)
        e = jnp.exp(lg - m)
        probs.append(e / jnp.sum(e, axis=0, keepdims=True))
    q1 = (probs[0][:, None, :] * probs[1][None, :, :]).reshape(100, _LANES)
    q2 = (probs[2][:, None, :] * probs[3][None, :, :]).reshape(100, _LANES)
    outr[...] = jnp.concatenate([q1, q2], axis=0)


def _run_net(X2, W1, b1, W2, b2, Wf1, bf1, Wf2, bf2, Wf3, bf3):
    m1 = jnp.dot(W1.reshape(6, 25), _S1).reshape(3456, 784)
    m1 = _aug(m1, jnp.repeat(b1, 576), 792)
    m2 = jnp.dot(W2.reshape(16, 150), _S2).reshape(1024, 864)
    m2 = _aug(m2, jnp.repeat(b2, 64), 872)
    f1 = _aug(Wf1, bf1, 264)
    f2 = _aug(Wf2, bf2, 128)
    f3 = _aug(Wf3, bf3, 88)
    full = lambda shape: pl.BlockSpec(shape, lambda i: (0,) * len(shape))
    return pl.pallas_call(
        _net_body,
        grid=(_B // _LANES,),
        in_specs=[
            pl.BlockSpec((3136, _LANES), lambda i: (0, i)),
            full((3456, 792)), full((1024, 872)),
            full((120, 264)), full((84, 128)), full((10, 88)),
        ],
        out_specs=pl.BlockSpec((_PK, _LANES), lambda i: (0, i)),
        out_shape=jax.ShapeDtypeStruct((_PK, _B), jnp.float32),
    )(X2, m1, m2, f1, f2, f3)


def _sc_body(d_hbm, out_hbm, dv, ov):
    wid = lax.axis_index("s") * 2 + lax.axis_index("c")
    for p in range(_NB // _NT):
        blk = wid * (_NB // _NT) + p
        pltpu.sync_copy(d_hbm.at[blk], dv)

        def vbody(v, _):
            jlo = jnp.maximum(0, v - 99)
            jhi1 = jnp.minimum(99, v) + 1

            def insert(j, M):
                x = dv[j] * dv[_Q2B + v - j]
                out = []
                for r in range(8):
                    mr = M[r]
                    out.append(jnp.maximum(mr, x))
                    x = jnp.minimum(mr, x)
                return tuple(out)

            M = lax.fori_loop(jlo, jhi1, insert,
                              (jnp.zeros((_SB,), jnp.float32),) * 8)
            s = M[0]
            for r in range(1, 8):
                s = s + M[r]
            ov[v] = s
            return 0

        lax.fori_loop(0, _NV, vbody, 0)
        pltpu.sync_copy(ov, out_hbm.at[blk])


def _run_sc(packed3):
    mesh = plsc.VectorSubcoreMesh(core_axis_name="c", subcore_axis_name="s")
    return pl.kernel(
        _sc_body,
        out_type=jax.ShapeDtypeStruct((_NB, _NV, _SB), jnp.float32),
        mesh=mesh,
        scratch_types=[
            pltpu.VMEM((_PK, _SB), jnp.float32),
            pltpu.VMEM((_NV, _SB), jnp.float32),
        ],
    )(packed3)


def _bucket_body(inr, outr):
    vv = lax.broadcasted_iota(jnp.int32, (_NV, 24), 0).astype(jnp.float32)
    cc = lax.broadcasted_iota(jnp.int32, (_NV, 24), 1).astype(jnp.float32)
    eqf = lambda a, b: jnp.maximum(0.0, 1.0 - jnp.abs(a - b))
    v10 = jnp.floor(vv / 10.0)
    d0 = vv - 10.0 * v10
    d2 = jnp.floor(vv / 100.0)
    d1 = v10 - 10.0 * d2
    ic = jnp.floor(cc / 10.0)
    digit = d0 * eqf(ic, 0.0) + d1 * eqf(ic, 1.0) + d2 * eqf(ic, 2.0)
    valid = jnp.clip(199.0 - vv, 0.0, 1.0)
    mask = eqf(digit, cc - 10.0 * ic) * valid
    outr[...] = jnp.dot(inr[...], mask, preferred_element_type=jnp.float32)


def _run_buckets(rows):
    return pl.pallas_call(
        _bucket_body,
        grid=(_B // _LANES,),
        in_specs=[pl.BlockSpec((_LANES, _NV), lambda i: (i, 0))],
        out_specs=pl.BlockSpec((_LANES, 24), lambda i: (i, 0)),
        out_shape=jax.ShapeDtypeStruct((_B, 24), jnp.float32),
    )(rows)


def kernel(x, W1, b1, W2, b2, Wf1, bf1, Wf2, bf2, Wf3, bf3):
    B = x.shape[0]
    X2 = x.reshape(B, 4 * 784).T
    packed = _run_net(X2, W1, b1, W2, b2, Wf1, bf1, Wf2, bf2, Wf3, bf3)
    packed3 = packed.reshape(_PK, _NB, _SB).transpose(1, 0, 2)
    rows = _run_sc(packed3).transpose(0, 2, 1).reshape(B, _NV)
    res = _run_buckets(rows)
    return (res[:, 0:10], res[:, 10:20], res[:, 20:22])


# speedup vs baseline: 97.0557x; 1.1455x over previous
import functools

import numpy as np
import jax
import jax.numpy as jnp
from jax import lax
from jax.experimental import pallas as pl
from jax.experimental.pallas import tpu as pltpu
from jax.experimental.pallas import tpu_sc as plsc

_B = 1024
_LANES = 128
_NT = 32
_SB = 16
_NB = _B // _SB
_PK = 200
_Q2B = 100
_NV = 208


def _conv_mat_indices(cout, cin, hin, win, k):
    hout, wout = hin - k + 1, win - k + 1
    o, i, j, c, di, dj = np.meshgrid(
        np.arange(cout), np.arange(hout), np.arange(wout),
        np.arange(cin), np.arange(k), np.arange(k), indexing="ij")
    ph = (i % 2) * 2 + (j % 2)
    rows = ((ph * cout + o) * (hout // 2) + i // 2) * (wout // 2) + j // 2
    cols = (c * hin + (i + di)) * win + (j + dj)
    widx = ((o * cin + c) * k + di) * k + dj
    return rows.ravel(), cols.ravel(), widx.ravel()

def _conv_sel(cin, hin, win, k):
    hout, wout = hin - k + 1, win - k + 1
    rows, cols, widx = _conv_mat_indices(1, cin, hin, win, k)
    s = np.zeros((cin * k * k, hout * wout * cin * hin * win), np.float32)
    s[widx, rows * (cin * hin * win) + cols] = 1.0
    return s

_S1 = _conv_sel(1, 28, 28, 5)
_S2 = _conv_sel(6, 12, 12, 5)


def _aug(mat, bias, pad_to):
    n, k = mat.shape
    return jnp.concatenate(
        [mat, bias.reshape(n, 1), jnp.zeros((n, pad_to - k - 1), mat.dtype)],
        axis=1)


def _ones_pad(v, pad_to):
    k, L = v.shape
    return jnp.concatenate(
        [v, jnp.ones((1, L), v.dtype), jnp.zeros((pad_to - k - 1, L), v.dtype)],
        axis=0)


def _pool_relu(h, n):
    t = jnp.maximum(jnp.maximum(h[0:n], h[n:2 * n]),
                    jnp.maximum(h[2 * n:3 * n], h[3 * n:4 * n]))
    return jnp.maximum(t, 0.0)


def _net_body(xr, m1r, m2r, f1r, f2r, f3r, outr):
    probs = []
    for d in range(4):
        Xa = _ones_pad(xr[784 * d:784 * (d + 1), :], 792)
        h1 = jnp.dot(m1r[...], Xa, preferred_element_type=jnp.float32)
        p1 = _pool_relu(h1, 864)
        h2 = jnp.dot(m2r[...], _ones_pad(p1, 872),
                     preferred_element_type=jnp.float32)
        p2 = _pool_relu(h2, 256)
        h = jnp.maximum(jnp.dot(f1r[...], _ones_pad(p2, 264),
                                preferred_element_type=jnp.float32), 0.0)
        h = jnp.maximum(jnp.dot(f2r[...], _ones_pad(h, 128),
                                preferred_element_type=jnp.float32), 0.0)
        lg = jnp.dot(f3r[...], _ones_pad(h, 88),
                     preferred_element_type=jnp.float32)
        m = jnp.max(lg, axis=0, keepdims=True)
        e = jnp.exp(lg - m)
        probs.append(e / jnp.sum(e, axis=0, keepdims=True))
    q1 = (probs[0][:, None, :] * probs[1][None, :, :]).reshape(100, _LANES)
    q2 = (probs[2][:, None, :] * probs[3][None, :, :]).reshape(100, _LANES)
    outr[...] = jnp.concatenate([q1, q2], axis=0)


def _run_net(X2, W1, b1, W2, b2, Wf1, bf1, Wf2, bf2, Wf3, bf3):
    m1 = jnp.dot(W1.reshape(6, 25), _S1).reshape(3456, 784)
    m1 = _aug(m1, jnp.tile(jnp.repeat(b1, 144), 4), 792)
    m2 = jnp.dot(W2.reshape(16, 150), _S2).reshape(1024, 864)
    m2 = _aug(m2, jnp.tile(jnp.repeat(b2, 16), 4), 872)
    f1 = _aug(Wf1, bf1, 264)
    f2 = _aug(Wf2, bf2, 128)
    f3 = _aug(Wf3, bf3, 88)
    full = lambda shape: pl.BlockSpec(shape, lambda i: (0,) * len(shape))
    return pl.pallas_call(
        _net_body,
        grid=(_B // _LANES,),
        in_specs=[
            pl.BlockSpec((3136, _LANES), lambda i: (0, i)),
            full((3456, 792)), full((1024, 872)),
            full((120, 264)), full((84, 128)), full((10, 88)),
        ],
        out_specs=pl.BlockSpec((_PK, _LANES), lambda i: (0, i)),
        out_shape=jax.ShapeDtypeStruct((_PK, _B), jnp.float32),
    )(X2, m1, m2, f1, f2, f3)


def _sc_body(d_hbm, out_hbm, dv, ov):
    wid = lax.axis_index("s") * 2 + lax.axis_index("c")
    for p in range(_NB // _NT):
        blk = wid * (_NB // _NT) + p
        pltpu.sync_copy(d_hbm.at[blk], dv)

        def vbody(v, _):
            jlo = jnp.maximum(0, v - 99)
            jhi1 = jnp.minimum(99, v) + 1

            def insert(j, M):
                x = dv[j] * dv[_Q2B + v - j]
                out = []
                for r in range(8):
                    mr = M[r]
                    out.append(jnp.maximum(mr, x))
                    x = jnp.minimum(mr, x)
                return tuple(out)

            M = lax.fori_loop(jlo, jhi1, insert,
                              (jnp.zeros((_SB,), jnp.float32),) * 8)
            s = M[0]
            for r in range(1, 8):
                s = s + M[r]
            ov[v] = s
            return 0

        lax.fori_loop(0, _NV, vbody, 0)
        pltpu.sync_copy(ov, out_hbm.at[blk])


def _run_sc(packed3):
    mesh = plsc.VectorSubcoreMesh(core_axis_name="c", subcore_axis_name="s")
    return pl.kernel(
        _sc_body,
        out_type=jax.ShapeDtypeStruct((_NB, _NV, _SB), jnp.float32),
        mesh=mesh,
        scratch_types=[
            pltpu.VMEM((_PK, _SB), jnp.float32),
            pltpu.VMEM((_NV, _SB), jnp.float32),
        ],
    )(packed3)


def _bucket_body(inr, outr):
    vv = lax.broadcasted_iota(jnp.int32, (_NV, 24), 0).astype(jnp.float32)
    cc = lax.broadcasted_iota(jnp.int32, (_NV, 24), 1).astype(jnp.float32)
    eqf = lambda a, b: jnp.maximum(0.0, 1.0 - jnp.abs(a - b))
    v10 = jnp.floor(vv / 10.0)
    d0 = vv - 10.0 * v10
    d2 = jnp.floor(vv / 100.0)
    d1 = v10 - 10.0 * d2
    ic = jnp.floor(cc / 10.0)
    digit = d0 * eqf(ic, 0.0) + d1 * eqf(ic, 1.0) + d2 * eqf(ic, 2.0)
    valid = jnp.clip(199.0 - vv, 0.0, 1.0)
    mask = eqf(digit, cc - 10.0 * ic) * valid
    outr[...] = jnp.dot(inr[...], mask, preferred_element_type=jnp.float32)


def _run_buckets(rows):
    return pl.pallas_call(
        _bucket_body,
        grid=(_B // _LANES,),
        in_specs=[pl.BlockSpec((_LANES, _NV), lambda i: (i, 0))],
        out_specs=pl.BlockSpec((_LANES, 24), lambda i: (i, 0)),
        out_shape=jax.ShapeDtypeStruct((_B, 24), jnp.float32),
    )(rows)


def kernel(x, W1, b1, W2, b2, Wf1, bf1, Wf2, bf2, Wf3, bf3):
    B = x.shape[0]
    X2 = x.reshape(B, 4 * 784).T
    packed = _run_net(X2, W1, b1, W2, b2, Wf1, bf1, Wf2, bf2, Wf3, bf3)
    packed3 = packed.reshape(_PK, _NB, _SB).transpose(1, 0, 2)
    rows = _run_sc(packed3).transpose(0, 2, 1).reshape(B, _NV)
    res = _run_buckets(rows)
    return (res[:, 0:10], res[:, 10:20], res[:, 20:22])


# speedup vs baseline: 99.4066x; 1.0242x over previous
import functools

import numpy as np
import jax
import jax.numpy as jnp
from jax import lax
from jax.experimental import pallas as pl
from jax.experimental.pallas import tpu as pltpu
from jax.experimental.pallas import tpu_sc as plsc

_B = 1024
_LANES = 128
_NT = 32
_SB = 16
_NB = _B // _SB
_PK = 200
_Q2B = 100
_NV = 208


def _conv_mat_indices(cout, cin, hin, win, k):
    hout, wout = hin - k + 1, win - k + 1
    o, i, j, c, di, dj = np.meshgrid(
        np.arange(cout), np.arange(hout), np.arange(wout),
        np.arange(cin), np.arange(k), np.arange(k), indexing="ij")
    ph = (i % 2) * 2 + (j % 2)
    rows = ((ph * cout + o) * (hout // 2) + i // 2) * (wout // 2) + j // 2
    cols = (c * hin + (i + di)) * win + (j + dj)
    widx = ((o * cin + c) * k + di) * k + dj
    return rows.ravel(), cols.ravel(), widx.ravel()

def _conv_sel(cin, hin, win, k):
    hout, wout = hin - k + 1, win - k + 1
    rows, cols, widx = _conv_mat_indices(1, cin, hin, win, k)
    s = np.zeros((cin * k * k, hout * wout * cin * hin * win), np.float32)
    s[widx, rows * (cin * hin * win) + cols] = 1.0
    return s

_S1 = _conv_sel(1, 28, 28, 5)
_S2 = _conv_sel(6, 12, 12, 5)


def _aug(mat, bias, pad_to):
    n, k = mat.shape
    return jnp.concatenate(
        [mat, bias.reshape(n, 1), jnp.zeros((n, pad_to - k - 1), mat.dtype)],
        axis=1)


def _ones_pad(v, pad_to):
    k, L = v.shape
    return jnp.concatenate(
        [v, jnp.ones((1, L), v.dtype), jnp.zeros((pad_to - k - 1, L), v.dtype)],
        axis=0)


def _pool_relu(h, c, m):
    t = jnp.max(h.reshape(c, 4, m, _LANES), axis=1)
    return jnp.maximum(t, 0.0).reshape(c * m, _LANES)


def _net_body(xr, m1r, m2r, f1r, f2r, f3r, outr):
    probs = []
    for d in range(4):
        Xa = _ones_pad(xr[784 * d:784 * (d + 1), :], 792)
        h1 = jnp.dot(m1r[...], Xa, preferred_element_type=jnp.float32)
        p1 = _pool_relu(h1, 6, 144)
        h2 = jnp.dot(m2r[...], _ones_pad(p1, 872),
                     preferred_element_type=jnp.float32)
        p2 = _pool_relu(h2, 16, 16)
        h = jnp.maximum(jnp.dot(f1r[...], _ones_pad(p2, 264),
                                preferred_element_type=jnp.float32), 0.0)
        h = jnp.maximum(jnp.dot(f2r[...], _ones_pad(h, 128),
                                preferred_element_type=jnp.float32), 0.0)
        lg = jnp.dot(f3r[...], _ones_pad(h, 88),
                     preferred_element_type=jnp.float32)
        m = jnp.max(lg, axis=0, keepdims=True)
        e = jnp.exp(lg - m)
        probs.append(e / jnp.sum(e, axis=0, keepdims=True))
    q1 = (probs[0][:, None, :] * probs[1][None, :, :]).reshape(100, _LANES)
    q2 = (probs[2][:, None, :] * probs[3][None, :, :]).reshape(100, _LANES)
    outr[...] = jnp.concatenate([q1, q2], axis=0)


def _run_net(X2, W1, b1, W2, b2, Wf1, bf1, Wf2, bf2, Wf3, bf3):
    m1 = jnp.dot(W1.reshape(6, 25), _S1).reshape(3456, 784)
    m1 = _aug(m1, jnp.repeat(b1, 576), 792)
    m2 = jnp.dot(W2.reshape(16, 150), _S2).reshape(1024, 864)
    m2 = _aug(m2, jnp.repeat(b2, 64), 872)
    f1 = _aug(Wf1, bf1, 264)
    f2 = _aug(Wf2, bf2, 128)
    f3 = _aug(Wf3, bf3, 88)
    full = lambda shape: pl.BlockSpec(shape, lambda i: (0,) * len(shape))
    return pl.pallas_call(
        _net_body,
        grid=(_B // _LANES,),
        in_specs=[
            pl.BlockSpec((3136, _LANES), lambda i: (0, i)),
            full((3456, 792)), full((1024, 872)),
            full((120, 264)), full((84, 128)), full((10, 88)),
        ],
        out_specs=pl.BlockSpec((_PK, _LANES), lambda i: (0, i)),
        out_shape=jax.ShapeDtypeStruct((_PK, _B), jnp.float32),
    )(X2, m1, m2, f1, f2, f3)


def _sc_body(d_hbm, out_hbm, dv, ov):
    wid = lax.axis_index("s") * 2 + lax.axis_index("c")
    for p in range(_NB // _NT):
        blk = wid * (_NB // _NT) + p
        pltpu.sync_copy(d_hbm.at[blk], dv)

        def vbody(v, _):
            jlo = jnp.maximum(0, v - 99)
            jhi1 = jnp.minimum(99, v) + 1

            def insert(j, M):
                x = dv[j] * dv[_Q2B + v - j]
                out = []
                for r in range(8):
                    mr = M[r]
                    out.append(jnp.maximum(mr, x))
                    x = jnp.minimum(mr, x)
                return tuple(out)

            M = lax.fori_loop(jlo, jhi1, insert,
                              (jnp.zeros((_SB,), jnp.float32),) * 8)
            s = M[0]
            for r in range(1, 8):
                s = s + M[r]
            ov[v] = s
            return 0

        lax.fori_loop(0, _NV, vbody, 0)
        pltpu.sync_copy(ov, out_hbm.at[blk])


def _run_sc(packed3):
    mesh = plsc.VectorSubcoreMesh(core_axis_name="c", subcore_axis_name="s")
    return pl.kernel(
        _sc_body,
        out_type=jax.ShapeDtypeStruct((_NB, _NV, _SB), jnp.float32),
        mesh=mesh,
        scratch_types=[
            pltpu.VMEM((_PK, _SB), jnp.float32),
            pltpu.VMEM((_NV, _SB), jnp.float32),
        ],
    )(packed3)


def _bucket_body(inr, outr):
    vv = lax.broadcasted_iota(jnp.int32, (_NV, 24), 0).astype(jnp.float32)
    cc = lax.broadcasted_iota(jnp.int32, (_NV, 24), 1).astype(jnp.float32)
    eqf = lambda a, b: jnp.maximum(0.0, 1.0 - jnp.abs(a - b))
    v10 = jnp.floor(vv / 10.0)
    d0 = vv - 10.0 * v10
    d2 = jnp.floor(vv / 100.0)
    d1 = v10 - 10.0 * d2
    ic = jnp.floor(cc / 10.0)
    digit = d0 * eqf(ic, 0.0) + d1 * eqf(ic, 1.0) + d2 * eqf(ic, 2.0)
    valid = jnp.clip(199.0 - vv, 0.0, 1.0)
    mask = eqf(digit, cc - 10.0 * ic) * valid
    outr[...] = jnp.dot(inr[...], mask, preferred_element_type=jnp.float32)


def _run_buckets(rows):
    return pl.pallas_call(
        _bucket_body,
        grid=(_B // _LANES,),
        in_specs=[pl.BlockSpec((_LANES, _NV), lambda i: (i, 0))],
        out_specs=pl.BlockSpec((_LANES, 24), lambda i: (i, 0)),
        out_shape=jax.ShapeDtypeStruct((_B, 24), jnp.float32),
    )(rows)


def kernel(x, W1, b1, W2, b2, Wf1, bf1, Wf2, bf2, Wf3, bf3):
    B = x.shape[0]
    X2 = x.reshape(B, 4 * 784).T
    packed = _run_net(X2, W1, b1, W2, b2, Wf1, bf1, Wf2, bf2, Wf3, bf3)
    packed3 = packed.reshape(_PK, _NB, _SB).transpose(1, 0, 2)
    rows = _run_sc(packed3).transpose(0, 2, 1).reshape(B, _NV)
    res = _run_buckets(rows)
    return (res[:, 0:10], res[:, 10:20], res[:, 20:22])
